# Initial kernel scaffold; baseline (speedup 1.0000x reference)
import jax
import jax.numpy as jnp
from jax.experimental import pallas as pl


def kernel(human_bt_n3, object_bt_m3, s_h_bt_n, s_o_bt_m, W1, b1, W2, b2):
    raise NotImplementedError("write your pallas kernel here")


# speedup vs baseline: 1.1474x; 1.1474x over previous
import functools

import jax
import jax.numpy as jnp
from jax.experimental import pallas as pl

_TAU = 0.05


def _feats_body(h_ref, o_ref, oT_ref, shc_ref, f_ref, *, nh, no, kqs):
    h = h_ref[0]
    o = o_ref[0]
    oT = oT_ref[0]
    shc = shc_ref[0]

    a2 = jnp.sum(h * h, axis=1, keepdims=True)
    b2 = jnp.sum(oT * oT, axis=0, keepdims=True)
    g = jnp.dot(h, oT, preferred_element_type=jnp.float32)
    sq = a2 + b2 - 2.0 * g
    d = jnp.sqrt(jnp.maximum(sq, 1e-12))

    dmin_h = jnp.min(d, axis=1, keepdims=True)
    dmin_o = jnp.min(d, axis=0, keepdims=True)

    iota_m = jax.lax.broadcasted_iota(jnp.int32, (nh, no), 1)
    idx = jnp.min(jnp.where(d == dmin_h, iota_m, no), axis=1, keepdims=True)
    onehot = (iota_m == idx).astype(jnp.float32)
    o_nn = jnp.dot(onehot, o, preferred_element_type=jnp.float32)

    vec = o_nn - h
    nrm = jnp.sqrt(jnp.maximum(jnp.sum(vec * vec, axis=1, keepdims=True), 1e-6))
    dir_mean = jnp.sum(vec / nrm, axis=0, keepdims=True) * (1.0 / nh)

    w_h = jnp.exp(dmin_h * (-1.0 / _TAU)) * shc

    dm_row = jnp.transpose(dmin_h)
    iota_r = jax.lax.broadcasted_iota(jnp.int32, (nh, nh), 0)
    iota_c = jax.lax.broadcasted_iota(jnp.int32, (nh, nh), 1)
    smaller = (dm_row < dmin_h) | ((dm_row == dmin_h) & (iota_c < iota_r))
    rank = jnp.dot(smaller.astype(jnp.float32),
                   jnp.ones((nh, 1), jnp.float32),
                   preferred_element_type=jnp.float32)

    qmeans = []
    for kq in kqs:
        sel = jnp.where(rank < kq, dmin_h, 0.0)
        qmeans.append(jnp.sum(sel, axis=0, keepdims=True) * (1.0 / kq))

    mean_dh = jnp.sum(dmin_h, axis=0, keepdims=True) * (1.0 / nh)
    min_dh = jnp.min(dmin_h, axis=0, keepdims=True)
    mean_wh = jnp.sum(w_h, axis=0, keepdims=True) * (1.0 / nh)
    mean_do = jnp.sum(dmin_o, axis=1, keepdims=True) * (1.0 / no)

    f_ref[...] = jnp.concatenate(
        [mean_dh, min_dh, qmeans[0], qmeans[1], qmeans[2],
         mean_wh, dir_mean, mean_do], axis=1)[None]


def _mlp_body(f_ref, w1_ref, b1_ref, w2_ref, b2_ref, out_ref):
    hid = jnp.maximum(
        jnp.dot(f_ref[...], w1_ref[...], preferred_element_type=jnp.float32)
        + b1_ref[...], 0.0)
    out_ref[...] = (
        jnp.dot(hid, w2_ref[...], preferred_element_type=jnp.float32)
        + b2_ref[...])


def kernel(human_bt_n3, object_bt_m3, s_h_bt_n, s_o_bt_m, W1, b1, W2, b2):
    B, T, Nh, _ = human_bt_n3.shape
    No = object_bt_m3.shape[2]
    BT = B * T
    h = human_bt_n3.reshape(BT, Nh, 3)
    o = object_bt_m3.reshape(BT, No, 3)
    oT = o.transpose(0, 2, 1)
    shc = s_h_bt_n.reshape(BT, Nh, 1)
    kqs = tuple(int(max(1, round(q * Nh))) for q in (0.2, 0.5, 0.8))

    feats = pl.pallas_call(
        functools.partial(_feats_body, nh=Nh, no=No, kqs=kqs),
        grid=(BT,),
        in_specs=[
            pl.BlockSpec((1, Nh, 3), lambda i: (i, 0, 0)),
            pl.BlockSpec((1, No, 3), lambda i: (i, 0, 0)),
            pl.BlockSpec((1, 3, No), lambda i: (i, 0, 0)),
            pl.BlockSpec((1, Nh, 1), lambda i: (i, 0, 0)),
        ],
        out_specs=pl.BlockSpec((1, 1, 10), lambda i: (i, 0, 0)),
        out_shape=jax.ShapeDtypeStruct((BT, 1, 10), jnp.float32),
    )(h, o, oT, shc)
    feats = feats.reshape(BT, 10)

    H = W1.shape[1]
    F = W2.shape[1]
    out = pl.pallas_call(
        _mlp_body,
        in_specs=[pl.BlockSpec(feats.shape, lambda: (0, 0)),
                  pl.BlockSpec(W1.shape, lambda: (0, 0)),
                  pl.BlockSpec((1, H), lambda: (0, 0)),
                  pl.BlockSpec(W2.shape, lambda: (0, 0)),
                  pl.BlockSpec((1, F), lambda: (0, 0))],
        out_specs=pl.BlockSpec((BT, F), lambda: (0, 0)),
        out_shape=jax.ShapeDtypeStruct((BT, F), jnp.float32),
    )(feats, W1, b1.reshape(1, H), W2, b2.reshape(1, F))
    return out.reshape(B, T, F)
